# baseline (speedup 1.0000x reference)
import functools

import jax
import jax.numpy as jnp
from jax import lax
from jax.experimental import pallas as pl
from jax.experimental.pallas import tpu as pltpu
from jax.experimental.pallas import tpu_sc as plsc

N = 10000
E = 320000
F = 128
FH = F // 2
NTILES = 16
CHUNK = 64
EPT = 20480
NCHUNKS = EPT // CHUNK
IBLK = 16
NBLOCKS = NCHUNKS // IBLK
GDEPTH = 4
EPAD = EPT * NTILES
NPAD = 10240
ROWS_PER_TILE = NPAD // NTILES


def _make_sc_body(with_deg):
    def body(x_hbm, src_hbm, dst_hbm, *rest):
        if with_deg:
            (agg_hbm, deg_hbm, x_sh, acc_sh, deg_sh, sbuf0, dbuf0,
             sbuf1, dbuf1, gbuf0, gbuf1, gbuf2, gbuf3, zdeg, ones_v,
             gsem, isem, ssem, dsem) = rest
        else:
            (agg_hbm, x_sh, acc_sh, sbuf0, dbuf0,
             sbuf1, dbuf1, gbuf0, gbuf1, gbuf2, gbuf3,
             gsem, isem, ssem) = rest
        gbufs = [gbuf0, gbuf1, gbuf2, gbuf3]
        cid = lax.axis_index("c")
        tid = lax.axis_index("s")

        def zrow(i, _):
            for c in range(4):
                gbuf0[i, pl.ds(c * 16, 16)] = jnp.zeros((16,), jnp.float32)
            return 0
        lax.fori_loop(0, CHUNK, zrow, 0)
        if with_deg:
            for c in range(ROWS_PER_TILE // 16):
                zdeg[pl.ds(c * 16, 16)] = jnp.zeros((16,), jnp.float32)
            for c in range(CHUNK // 16):
                ones_v[pl.ds(c * 16, 16)] = jnp.ones((16,), jnp.float32)

        for b in range(ROWS_PER_TILE // CHUNK):
            pltpu.sync_copy(gbuf0, acc_sh.at[pl.ds(tid * ROWS_PER_TILE
                                                   + b * CHUNK, CHUNK)])
        if with_deg:
            pltpu.sync_copy(zdeg, deg_sh.at[pl.ds(tid * ROWS_PER_TILE,
                                                  ROWS_PER_TILE)])

        pltpu.sync_copy(
            x_hbm.at[cid, pl.ds(tid * ROWS_PER_TILE, ROWS_PER_TILE)],
            x_sh.at[pl.ds(tid * ROWS_PER_TILE, ROWS_PER_TILE)])

        pltpu.sync_copy(src_hbm.at[tid, pl.ds(0, IBLK)], sbuf0)
        pltpu.sync_copy(dst_hbm.at[tid, pl.ds(0, IBLK)], dbuf0)

        plsc.subcore_barrier()

        for k in range(GDEPTH):
            pltpu.async_copy(x_sh.at[sbuf0.at[k]], gbufs[k], gsem)

        for blk in range(NBLOCKS):
            pS, pD = (sbuf0, dbuf0) if blk % 2 == 0 else (sbuf1, dbuf1)
            nS, nD = (sbuf1, dbuf1) if blk % 2 == 0 else (sbuf0, dbuf0)
            if blk + 1 < NBLOCKS:
                ips = pltpu.async_copy(
                    src_hbm.at[tid, pl.ds((blk + 1) * IBLK, IBLK)], nS, isem)
                ipd = pltpu.async_copy(
                    dst_hbm.at[tid, pl.ds((blk + 1) * IBLK, IBLK)], nD, isem)

            def quad(s, _):
                c = GDEPTH * s
                for k in range(GDEPTH):
                    pltpu.make_async_copy(
                        x_sh.at[pS.at[c + k]], gbufs[k], gsem).wait()
                    pltpu.async_copy(gbufs[k], acc_sh.at[pD.at[c + k]], ssem,
                                     add=True)
                    if with_deg:
                        pltpu.async_copy(ones_v, deg_sh.at[pD.at[c + k]],
                                         dsem, add=True)
                for k in range(GDEPTH):
                    pltpu.make_async_copy(
                        gbufs[k], acc_sh.at[pD.at[c + k]], ssem).wait()

                    @pl.when(s < IBLK // GDEPTH - 1)
                    def _(k=k):
                        pltpu.async_copy(x_sh.at[pS.at[c + GDEPTH + k]],
                                         gbufs[k], gsem)
                return 0
            lax.fori_loop(0, IBLK // GDEPTH, quad, 0)

            if with_deg:
                def ddrain(i, _):
                    pltpu.make_async_copy(
                        ones_v, deg_sh.at[pD.at[0]], dsem).wait()
                    return 0
                lax.fori_loop(0, IBLK, ddrain, 0)

            if blk + 1 < NBLOCKS:
                ips.wait()
                ipd.wait()
                for k in range(GDEPTH):
                    pltpu.async_copy(x_sh.at[nS.at[k]], gbufs[k], gsem)

        plsc.subcore_barrier()

        pltpu.sync_copy(
            acc_sh.at[pl.ds(tid * ROWS_PER_TILE, ROWS_PER_TILE)],
            agg_hbm.at[cid, pl.ds(tid * ROWS_PER_TILE, ROWS_PER_TILE)])

        if with_deg:
            @pl.when(cid == 0)
            def _():
                pltpu.sync_copy(
                    deg_sh.at[pl.ds(tid * ROWS_PER_TILE, ROWS_PER_TILE)],
                    deg_hbm.at[pl.ds(tid * ROWS_PER_TILE, ROWS_PER_TILE)])
    return body


def _make_sc_agg(with_deg):
    agg_t = jax.ShapeDtypeStruct((2, NPAD, FH), jnp.float32)
    deg_t = jax.ShapeDtypeStruct((NPAD,), jnp.float32)
    scratch = [
        pltpu.VMEM_SHARED((NPAD, FH), jnp.float32),
        pltpu.VMEM_SHARED((NPAD, FH), jnp.float32),
    ]
    if with_deg:
        scratch.append(pltpu.VMEM_SHARED((NPAD,), jnp.float32))
    scratch += [
        pltpu.VMEM((IBLK, CHUNK), jnp.int32),
        pltpu.VMEM((IBLK, CHUNK), jnp.int32),
        pltpu.VMEM((IBLK, CHUNK), jnp.int32),
        pltpu.VMEM((IBLK, CHUNK), jnp.int32),
        pltpu.VMEM((CHUNK, FH), jnp.float32),
        pltpu.VMEM((CHUNK, FH), jnp.float32),
        pltpu.VMEM((CHUNK, FH), jnp.float32),
        pltpu.VMEM((CHUNK, FH), jnp.float32),
    ]
    if with_deg:
        scratch += [
            pltpu.VMEM((ROWS_PER_TILE,), jnp.float32),
            pltpu.VMEM((CHUNK,), jnp.float32),
        ]
    scratch += [pltpu.SemaphoreType.DMA, pltpu.SemaphoreType.DMA,
                pltpu.SemaphoreType.DMA]
    if with_deg:
        scratch.append(pltpu.SemaphoreType.DMA)
    return pl.kernel(
        _make_sc_body(with_deg),
        out_type=(agg_t, deg_t) if with_deg else agg_t,
        mesh=plsc.VectorSubcoreMesh(core_axis_name="c", subcore_axis_name="s"),
        scratch_types=scratch,
    )


_sc_agg_deg = _make_sc_agg(True)
_sc_agg_nodeg = _make_sc_agg(False)


def _mm1_body(aggs_ref, deg_ref, x_ref, wl_ref, wr_ref, b_ref, out_ref):
    agg = jnp.concatenate([aggs_ref[0], aggs_ref[1]], axis=1)
    d = jnp.maximum(deg_ref[:], 1.0)
    a = agg / d
    z = (lax.dot_general(a, wl_ref[:], (((1,), (1,)), ((), ())),
                         preferred_element_type=jnp.float32)
         + lax.dot_general(x_ref[:], wr_ref[:], (((1,), (1,)), ((), ())),
                           preferred_element_type=jnp.float32)
         + b_ref[:])
    h = jnp.maximum(z, 0.0)
    out_ref[0] = h[:, :FH]
    out_ref[1] = h[:, FH:]


def _mm2_body(aggs_ref, deg_ref, hs_ref, wl_ref, wr_ref, b_ref,
              wlin_ref, blin_ref, out_ref):
    agg = jnp.concatenate([aggs_ref[0], aggs_ref[1]], axis=1)
    hprev = jnp.concatenate([hs_ref[0], hs_ref[1]], axis=1)
    d = jnp.maximum(deg_ref[:], 1.0)
    a = agg / d
    z = (lax.dot_general(a, wl_ref[:], (((1,), (1,)), ((), ())),
                         preferred_element_type=jnp.float32)
         + lax.dot_general(hprev, wr_ref[:], (((1,), (1,)), ((), ())),
                           preferred_element_type=jnp.float32)
         + b_ref[:])
    h = jnp.maximum(z, 0.0)
    out_ref[:] = (lax.dot_general(h, wlin_ref[:], (((1,), (1,)), ((), ())),
                                  preferred_element_type=jnp.float32)
                  + blin_ref[0])


_BLK = 1024
_GRID = NPAD // _BLK


def _mm1(aggs, deg, x, wl, wr, b):
    return pl.pallas_call(
        _mm1_body,
        grid=(_GRID,),
        in_specs=[
            pl.BlockSpec((2, _BLK, FH), lambda i: (0, i, 0)),
            pl.BlockSpec((_BLK, 1), lambda i: (i, 0)),
            pl.BlockSpec((_BLK, F), lambda i: (i, 0)),
            pl.BlockSpec((F, F), lambda i: (0, 0)),
            pl.BlockSpec((F, F), lambda i: (0, 0)),
            pl.BlockSpec((1, F), lambda i: (0, 0)),
        ],
        out_specs=pl.BlockSpec((2, _BLK, FH), lambda i: (0, i, 0)),
        out_shape=jax.ShapeDtypeStruct((2, NPAD, FH), jnp.float32),
    )(aggs, deg, x, wl, wr, b)


def _mm2(aggs, deg, hs, wl, wr, b, wlin, blin):
    return pl.pallas_call(
        _mm2_body,
        grid=(_GRID,),
        in_specs=[
            pl.BlockSpec((2, _BLK, FH), lambda i: (0, i, 0)),
            pl.BlockSpec((_BLK, 1), lambda i: (i, 0)),
            pl.BlockSpec((2, _BLK, FH), lambda i: (0, i, 0)),
            pl.BlockSpec((F, F), lambda i: (0, 0)),
            pl.BlockSpec((F, F), lambda i: (0, 0)),
            pl.BlockSpec((1, F), lambda i: (0, 0)),
            pl.BlockSpec((F, F), lambda i: (0, 0)),
            pl.BlockSpec(memory_space=pltpu.SMEM),
        ],
        out_specs=pl.BlockSpec((_BLK, F), lambda i: (i, 0)),
        out_shape=jax.ShapeDtypeStruct((NPAD, F), jnp.float32),
    )(aggs, deg, hs, wl, wr, b, wlin, blin)


def kernel(x, edge_index, W1l, b1, W1r, W2l, b2, W2r, Wlin, blin):
    src = edge_index[0]
    dst = edge_index[1]
    npad = EPAD - E
    src_p = jnp.concatenate([src, jnp.zeros((npad,), jnp.int32)])
    dst_p = jnp.concatenate(
        [dst, N + (jnp.arange(npad, dtype=jnp.int32) % (NPAD - N))])
    src_r = src_p.reshape(NTILES, NCHUNKS, CHUNK)
    dst_r = dst_p.reshape(NTILES, NCHUNKS, CHUNK)

    xp = jnp.pad(x, ((0, NPAD - N), (0, 0)))
    xs = jnp.stack([xp[:, :FH], xp[:, FH:]])

    agg1, deg = _sc_agg_deg(xs, src_r, dst_r)
    degc = deg[:, None]

    h1s = _mm1(agg1, degc, xp, W1l, W1r, b1.reshape(1, F))

    agg2 = _sc_agg_nodeg(h1s, src_r, dst_r)

    wlinp = jnp.pad(Wlin, ((0, F - 1), (0, 0)))
    out = _mm2(agg2, degc, h1s, W2l, W2r, b2.reshape(1, F),
               wlinp, blin)
    return out[:N, :1]


# speedup vs baseline: 1.0524x; 1.0524x over previous
import functools

import jax
import jax.numpy as jnp
from jax import lax
from jax.experimental import pallas as pl
from jax.experimental.pallas import tpu as pltpu
from jax.experimental.pallas import tpu_sc as plsc

N = 10000
E = 320000
F = 128
FH = F // 2
NTILES = 16
CHUNK = 128
EPT = 20480
NCHUNKS = EPT // CHUNK
IBLK = 16
NBLOCKS = NCHUNKS // IBLK
EPAD = EPT * NTILES
NPAD = 10240
ROWS_PER_TILE = NPAD // NTILES


def _make_sc_body(with_deg):
    def body(x_hbm, src_hbm, dst_hbm, *rest):
        if with_deg:
            (agg_hbm, deg_hbm, x_sh, acc_sh, deg_sh, sbuf0, dbuf0,
             sbuf1, dbuf1, gbuf0, gbuf1, zdeg, ones_v,
             gsem, isem, ssem, dsem) = rest
        else:
            (agg_hbm, x_sh, acc_sh, sbuf0, dbuf0,
             sbuf1, dbuf1, gbuf0, gbuf1, gsem, isem, ssem) = rest
        cid = lax.axis_index("c")
        tid = lax.axis_index("s")

        def zrow(i, _):
            for c in range(4):
                gbuf0[i, pl.ds(c * 16, 16)] = jnp.zeros((16,), jnp.float32)
            return 0
        lax.fori_loop(0, CHUNK, zrow, 0)
        if with_deg:
            for c in range(ROWS_PER_TILE // 16):
                zdeg[pl.ds(c * 16, 16)] = jnp.zeros((16,), jnp.float32)
            for c in range(CHUNK // 16):
                ones_v[pl.ds(c * 16, 16)] = jnp.ones((16,), jnp.float32)

        for b in range(ROWS_PER_TILE // CHUNK):
            pltpu.sync_copy(gbuf0, acc_sh.at[pl.ds(tid * ROWS_PER_TILE
                                                   + b * CHUNK, CHUNK)])
        if with_deg:
            pltpu.sync_copy(zdeg, deg_sh.at[pl.ds(tid * ROWS_PER_TILE,
                                                  ROWS_PER_TILE)])

        pltpu.sync_copy(
            x_hbm.at[cid, pl.ds(tid * ROWS_PER_TILE, ROWS_PER_TILE)],
            x_sh.at[pl.ds(tid * ROWS_PER_TILE, ROWS_PER_TILE)])

        pltpu.sync_copy(src_hbm.at[tid, pl.ds(0, IBLK)], sbuf0)
        pltpu.sync_copy(dst_hbm.at[tid, pl.ds(0, IBLK)], dbuf0)

        plsc.subcore_barrier()

        pltpu.async_copy(x_sh.at[sbuf0.at[0]], gbuf0, gsem)
        pltpu.async_copy(x_sh.at[sbuf0.at[1]], gbuf1, gsem)

        for blk in range(NBLOCKS):
            pS, pD = (sbuf0, dbuf0) if blk % 2 == 0 else (sbuf1, dbuf1)
            nS, nD = (sbuf1, dbuf1) if blk % 2 == 0 else (sbuf0, dbuf0)
            if blk + 1 < NBLOCKS:
                ips = pltpu.async_copy(
                    src_hbm.at[tid, pl.ds((blk + 1) * IBLK, IBLK)], nS, isem)
                ipd = pltpu.async_copy(
                    dst_hbm.at[tid, pl.ds((blk + 1) * IBLK, IBLK)], nD, isem)

            deg_core = 0 if blk < NBLOCKS // 2 else 1

            def pair(s, _):
                c = 2 * s
                pltpu.make_async_copy(x_sh.at[pS.at[c]], gbuf0, gsem).wait()
                pltpu.async_copy(gbuf0, acc_sh.at[pD.at[c]], ssem, add=True)
                if with_deg:
                    @pl.when(cid == deg_core)
                    def _():
                        pltpu.async_copy(ones_v, deg_sh.at[pD.at[c]], dsem,
                                         add=True)
                pltpu.make_async_copy(
                    x_sh.at[pS.at[c + 1]], gbuf1, gsem).wait()
                pltpu.async_copy(gbuf1, acc_sh.at[pD.at[c + 1]], ssem,
                                 add=True)
                if with_deg:
                    @pl.when(cid == deg_core)
                    def _():
                        pltpu.async_copy(ones_v, deg_sh.at[pD.at[c + 1]],
                                         dsem, add=True)
                pltpu.make_async_copy(gbuf0, acc_sh.at[pD.at[c]], ssem).wait()

                @pl.when(s < IBLK // 2 - 1)
                def _():
                    pltpu.async_copy(x_sh.at[pS.at[c + 2]], gbuf0, gsem)
                pltpu.make_async_copy(
                    gbuf1, acc_sh.at[pD.at[c + 1]], ssem).wait()

                @pl.when(s < IBLK // 2 - 1)
                def _():
                    pltpu.async_copy(x_sh.at[pS.at[c + 3]], gbuf1, gsem)
                return 0
            lax.fori_loop(0, IBLK // 2, pair, 0)

            if with_deg:
                @pl.when(cid == deg_core)
                def _():
                    def ddrain(i, _):
                        pltpu.make_async_copy(
                            ones_v, deg_sh.at[pD.at[0]], dsem).wait()
                        return 0
                    lax.fori_loop(0, IBLK, ddrain, 0)

            if blk + 1 < NBLOCKS:
                ips.wait()
                ipd.wait()
                pltpu.async_copy(x_sh.at[nS.at[0]], gbuf0, gsem)
                pltpu.async_copy(x_sh.at[nS.at[1]], gbuf1, gsem)

        plsc.subcore_barrier()

        pltpu.sync_copy(
            acc_sh.at[pl.ds(tid * ROWS_PER_TILE, ROWS_PER_TILE)],
            agg_hbm.at[cid, pl.ds(tid * ROWS_PER_TILE, ROWS_PER_TILE)])

        if with_deg:
            pltpu.sync_copy(
                deg_sh.at[pl.ds(tid * ROWS_PER_TILE, ROWS_PER_TILE)],
                deg_hbm.at[cid, pl.ds(tid * ROWS_PER_TILE, ROWS_PER_TILE)])
    return body


def _make_sc_agg(with_deg):
    agg_t = jax.ShapeDtypeStruct((2, NPAD, FH), jnp.float32)
    deg_t = jax.ShapeDtypeStruct((2, NPAD), jnp.float32)
    scratch = [
        pltpu.VMEM_SHARED((NPAD, FH), jnp.float32),
        pltpu.VMEM_SHARED((NPAD, FH), jnp.float32),
    ]
    if with_deg:
        scratch.append(pltpu.VMEM_SHARED((NPAD,), jnp.float32))
    scratch += [
        pltpu.VMEM((IBLK, CHUNK), jnp.int32),
        pltpu.VMEM((IBLK, CHUNK), jnp.int32),
        pltpu.VMEM((IBLK, CHUNK), jnp.int32),
        pltpu.VMEM((IBLK, CHUNK), jnp.int32),
        pltpu.VMEM((CHUNK, FH), jnp.float32),
        pltpu.VMEM((CHUNK, FH), jnp.float32),
    ]
    if with_deg:
        scratch += [
            pltpu.VMEM((ROWS_PER_TILE,), jnp.float32),
            pltpu.VMEM((CHUNK,), jnp.float32),
        ]
    scratch += [pltpu.SemaphoreType.DMA, pltpu.SemaphoreType.DMA,
                pltpu.SemaphoreType.DMA]
    if with_deg:
        scratch.append(pltpu.SemaphoreType.DMA)
    return pl.kernel(
        _make_sc_body(with_deg),
        out_type=(agg_t, deg_t) if with_deg else agg_t,
        mesh=plsc.VectorSubcoreMesh(core_axis_name="c", subcore_axis_name="s"),
        scratch_types=scratch,
    )


_sc_agg_deg = _make_sc_agg(True)
_sc_agg_nodeg = _make_sc_agg(False)


def _mm1_body(aggs_ref, deg_ref, x_ref, wl_ref, wr_ref, b_ref, out_ref):
    agg = jnp.concatenate([aggs_ref[0], aggs_ref[1]], axis=1)
    d = jnp.maximum(deg_ref[0] + deg_ref[1], 1.0)
    a = agg / d
    z = (lax.dot_general(a, wl_ref[:], (((1,), (1,)), ((), ())),
                         preferred_element_type=jnp.float32)
         + lax.dot_general(x_ref[:], wr_ref[:], (((1,), (1,)), ((), ())),
                           preferred_element_type=jnp.float32)
         + b_ref[:])
    h = jnp.maximum(z, 0.0)
    out_ref[0] = h[:, :FH]
    out_ref[1] = h[:, FH:]


def _mm2_body(aggs_ref, deg_ref, hs_ref, wl_ref, wr_ref, b_ref,
              wlin_ref, blin_ref, out_ref):
    agg = jnp.concatenate([aggs_ref[0], aggs_ref[1]], axis=1)
    hprev = jnp.concatenate([hs_ref[0], hs_ref[1]], axis=1)
    d = jnp.maximum(deg_ref[0] + deg_ref[1], 1.0)
    a = agg / d
    z = (lax.dot_general(a, wl_ref[:], (((1,), (1,)), ((), ())),
                         preferred_element_type=jnp.float32)
         + lax.dot_general(hprev, wr_ref[:], (((1,), (1,)), ((), ())),
                           preferred_element_type=jnp.float32)
         + b_ref[:])
    h = jnp.maximum(z, 0.0)
    out_ref[:] = (lax.dot_general(h, wlin_ref[:], (((1,), (1,)), ((), ())),
                                  preferred_element_type=jnp.float32)
                  + blin_ref[0])


_BLK = 1024
_GRID = NPAD // _BLK


def _mm1(aggs, deg, x, wl, wr, b):
    return pl.pallas_call(
        _mm1_body,
        grid=(_GRID,),
        in_specs=[
            pl.BlockSpec((2, _BLK, FH), lambda i: (0, i, 0)),
            pl.BlockSpec((2, _BLK, 1), lambda i: (0, i, 0)),
            pl.BlockSpec((_BLK, F), lambda i: (i, 0)),
            pl.BlockSpec((F, F), lambda i: (0, 0)),
            pl.BlockSpec((F, F), lambda i: (0, 0)),
            pl.BlockSpec((1, F), lambda i: (0, 0)),
        ],
        out_specs=pl.BlockSpec((2, _BLK, FH), lambda i: (0, i, 0)),
        out_shape=jax.ShapeDtypeStruct((2, NPAD, FH), jnp.float32),
    )(aggs, deg, x, wl, wr, b)


def _mm2(aggs, deg, hs, wl, wr, b, wlin, blin):
    return pl.pallas_call(
        _mm2_body,
        grid=(_GRID,),
        in_specs=[
            pl.BlockSpec((2, _BLK, FH), lambda i: (0, i, 0)),
            pl.BlockSpec((2, _BLK, 1), lambda i: (0, i, 0)),
            pl.BlockSpec((2, _BLK, FH), lambda i: (0, i, 0)),
            pl.BlockSpec((F, F), lambda i: (0, 0)),
            pl.BlockSpec((F, F), lambda i: (0, 0)),
            pl.BlockSpec((1, F), lambda i: (0, 0)),
            pl.BlockSpec((F, F), lambda i: (0, 0)),
            pl.BlockSpec(memory_space=pltpu.SMEM),
        ],
        out_specs=pl.BlockSpec((_BLK, F), lambda i: (i, 0)),
        out_shape=jax.ShapeDtypeStruct((NPAD, F), jnp.float32),
    )(aggs, deg, hs, wl, wr, b, wlin, blin)


def kernel(x, edge_index, W1l, b1, W1r, W2l, b2, W2r, Wlin, blin):
    src = edge_index[0]
    dst = edge_index[1]
    npad = EPAD - E
    src_p = jnp.concatenate([src, jnp.zeros((npad,), jnp.int32)])
    dst_p = jnp.concatenate(
        [dst, N + (jnp.arange(npad, dtype=jnp.int32) % (NPAD - N))])
    src_r = src_p.reshape(NTILES, NCHUNKS, CHUNK)
    dst_r = dst_p.reshape(NTILES, NCHUNKS, CHUNK)

    xp = jnp.pad(x, ((0, NPAD - N), (0, 0)))
    xs = jnp.stack([xp[:, :FH], xp[:, FH:]])

    agg1, deg = _sc_agg_deg(xs, src_r, dst_r)
    degc = deg[:, :, None]

    h1s = _mm1(agg1, degc, xp, W1l, W1r, b1.reshape(1, F))

    agg2 = _sc_agg_nodeg(h1s, src_r, dst_r)

    wlinp = jnp.pad(Wlin, ((0, F - 1), (0, 0)))
    out = _mm2(agg2, degc, h1s, W2l, W2r, b2.reshape(1, F),
               wlinp, blin)
    return out[:N, :1]


# speedup vs baseline: 1.0538x; 1.0014x over previous
import functools

import jax
import jax.numpy as jnp
from jax import lax
from jax.experimental import pallas as pl
from jax.experimental.pallas import tpu as pltpu
from jax.experimental.pallas import tpu_sc as plsc

N = 10000
E = 320000
F = 128
FH = F // 2
NTILES = 16
CHUNK = 128
EPT = 20480
NCHUNKS = EPT // CHUNK
IBLK = 16
NBLOCKS = NCHUNKS // IBLK
GDEPTH = 4
EPAD = EPT * NTILES
NPAD = 10240
ROWS_PER_TILE = NPAD // NTILES


def _make_sc_body(with_deg):
    def body(x_hbm, src_hbm, dst_hbm, *rest):
        if with_deg:
            (agg_hbm, deg_hbm, x_sh, acc_sh, deg_sh, sbuf0, dbuf0,
             sbuf1, dbuf1, gbuf0, gbuf1, gbuf2, gbuf3, zdeg, ones_v,
             gsem, isem, ssem, dsem) = rest
        else:
            (agg_hbm, x_sh, acc_sh, sbuf0, dbuf0,
             sbuf1, dbuf1, gbuf0, gbuf1, gbuf2, gbuf3,
             gsem, isem, ssem) = rest
        gbufs = [gbuf0, gbuf1, gbuf2, gbuf3]
        cid = lax.axis_index("c")
        tid = lax.axis_index("s")

        def zrow(i, _):
            for c in range(4):
                gbuf0[i, pl.ds(c * 16, 16)] = jnp.zeros((16,), jnp.float32)
            return 0
        lax.fori_loop(0, CHUNK, zrow, 0)
        if with_deg:
            for c in range(ROWS_PER_TILE // 16):
                zdeg[pl.ds(c * 16, 16)] = jnp.zeros((16,), jnp.float32)
            for c in range(CHUNK // 16):
                ones_v[pl.ds(c * 16, 16)] = jnp.ones((16,), jnp.float32)

        for b in range(ROWS_PER_TILE // CHUNK):
            pltpu.sync_copy(gbuf0, acc_sh.at[pl.ds(tid * ROWS_PER_TILE
                                                   + b * CHUNK, CHUNK)])
        if with_deg:
            pltpu.sync_copy(zdeg, deg_sh.at[pl.ds(tid * ROWS_PER_TILE,
                                                  ROWS_PER_TILE)])

        pltpu.sync_copy(
            x_hbm.at[cid, pl.ds(tid * ROWS_PER_TILE, ROWS_PER_TILE)],
            x_sh.at[pl.ds(tid * ROWS_PER_TILE, ROWS_PER_TILE)])

        pltpu.sync_copy(src_hbm.at[tid, pl.ds(0, IBLK)], sbuf0)
        pltpu.sync_copy(dst_hbm.at[tid, pl.ds(0, IBLK)], dbuf0)

        plsc.subcore_barrier()

        for k in range(GDEPTH):
            pltpu.async_copy(x_sh.at[sbuf0.at[k]], gbufs[k], gsem)

        for blk in range(NBLOCKS):
            pS, pD = (sbuf0, dbuf0) if blk % 2 == 0 else (sbuf1, dbuf1)
            nS, nD = (sbuf1, dbuf1) if blk % 2 == 0 else (sbuf0, dbuf0)
            if blk + 1 < NBLOCKS:
                ips = pltpu.async_copy(
                    src_hbm.at[tid, pl.ds((blk + 1) * IBLK, IBLK)], nS, isem)
                ipd = pltpu.async_copy(
                    dst_hbm.at[tid, pl.ds((blk + 1) * IBLK, IBLK)], nD, isem)

            deg_core = 0 if blk < NBLOCKS // 2 else 1

            def quad(s, _):
                c = GDEPTH * s
                for k in range(GDEPTH):
                    pltpu.make_async_copy(
                        x_sh.at[pS.at[c + k]], gbufs[k], gsem).wait()
                    pltpu.async_copy(gbufs[k], acc_sh.at[pD.at[c + k]], ssem,
                                     add=True)
                    if with_deg:
                        @pl.when(cid == deg_core)
                        def _(k=k):
                            pltpu.async_copy(ones_v, deg_sh.at[pD.at[c + k]],
                                             dsem, add=True)
                for k in range(GDEPTH):
                    pltpu.make_async_copy(
                        gbufs[k], acc_sh.at[pD.at[c + k]], ssem).wait()

                    @pl.when(s < IBLK // GDEPTH - 1)
                    def _(k=k):
                        pltpu.async_copy(x_sh.at[pS.at[c + GDEPTH + k]],
                                         gbufs[k], gsem)
                return 0
            lax.fori_loop(0, IBLK // GDEPTH, quad, 0)

            if with_deg:
                @pl.when(cid == deg_core)
                def _():
                    def ddrain(i, _):
                        pltpu.make_async_copy(
                            ones_v, deg_sh.at[pD.at[0]], dsem).wait()
                        return 0
                    lax.fori_loop(0, IBLK, ddrain, 0)

            if blk + 1 < NBLOCKS:
                ips.wait()
                ipd.wait()
                for k in range(GDEPTH):
                    pltpu.async_copy(x_sh.at[nS.at[k]], gbufs[k], gsem)

        plsc.subcore_barrier()

        pltpu.sync_copy(
            acc_sh.at[pl.ds(tid * ROWS_PER_TILE, ROWS_PER_TILE)],
            agg_hbm.at[cid, pl.ds(tid * ROWS_PER_TILE, ROWS_PER_TILE)])

        if with_deg:
            pltpu.sync_copy(
                deg_sh.at[pl.ds(tid * ROWS_PER_TILE, ROWS_PER_TILE)],
                deg_hbm.at[cid, pl.ds(tid * ROWS_PER_TILE, ROWS_PER_TILE)])
    return body


def _make_sc_agg(with_deg):
    agg_t = jax.ShapeDtypeStruct((2, NPAD, FH), jnp.float32)
    deg_t = jax.ShapeDtypeStruct((2, NPAD), jnp.float32)
    scratch = [
        pltpu.VMEM_SHARED((NPAD, FH), jnp.float32),
        pltpu.VMEM_SHARED((NPAD, FH), jnp.float32),
    ]
    if with_deg:
        scratch.append(pltpu.VMEM_SHARED((NPAD,), jnp.float32))
    scratch += [
        pltpu.VMEM((IBLK, CHUNK), jnp.int32),
        pltpu.VMEM((IBLK, CHUNK), jnp.int32),
        pltpu.VMEM((IBLK, CHUNK), jnp.int32),
        pltpu.VMEM((IBLK, CHUNK), jnp.int32),
        pltpu.VMEM((CHUNK, FH), jnp.float32),
        pltpu.VMEM((CHUNK, FH), jnp.float32),
        pltpu.VMEM((CHUNK, FH), jnp.float32),
        pltpu.VMEM((CHUNK, FH), jnp.float32),
    ]
    if with_deg:
        scratch += [
            pltpu.VMEM((ROWS_PER_TILE,), jnp.float32),
            pltpu.VMEM((CHUNK,), jnp.float32),
        ]
    scratch += [pltpu.SemaphoreType.DMA, pltpu.SemaphoreType.DMA,
                pltpu.SemaphoreType.DMA]
    if with_deg:
        scratch.append(pltpu.SemaphoreType.DMA)
    return pl.kernel(
        _make_sc_body(with_deg),
        out_type=(agg_t, deg_t) if with_deg else agg_t,
        mesh=plsc.VectorSubcoreMesh(core_axis_name="c", subcore_axis_name="s"),
        scratch_types=scratch,
        compiler_params=pltpu.CompilerParams(use_tc_tiling_on_sc=False),
    )


_sc_agg_deg = _make_sc_agg(True)
_sc_agg_nodeg = _make_sc_agg(False)


def _mm1_body(aggs_ref, deg_ref, x_ref, wl_ref, wr_ref, b_ref, out_ref):
    agg = jnp.concatenate([aggs_ref[0], aggs_ref[1]], axis=1)
    d = jnp.maximum(deg_ref[0] + deg_ref[1], 1.0)
    a = agg / d
    z = (lax.dot_general(a, wl_ref[:], (((1,), (1,)), ((), ())),
                         preferred_element_type=jnp.float32)
         + lax.dot_general(x_ref[:], wr_ref[:], (((1,), (1,)), ((), ())),
                           preferred_element_type=jnp.float32)
         + b_ref[:])
    h = jnp.maximum(z, 0.0)
    out_ref[0] = h[:, :FH]
    out_ref[1] = h[:, FH:]


def _mm2_body(aggs_ref, deg_ref, hs_ref, wl_ref, wr_ref, b_ref,
              wlin_ref, blin_ref, out_ref):
    agg = jnp.concatenate([aggs_ref[0], aggs_ref[1]], axis=1)
    hprev = jnp.concatenate([hs_ref[0], hs_ref[1]], axis=1)
    d = jnp.maximum(deg_ref[0] + deg_ref[1], 1.0)
    a = agg / d
    z = (lax.dot_general(a, wl_ref[:], (((1,), (1,)), ((), ())),
                         preferred_element_type=jnp.float32)
         + lax.dot_general(hprev, wr_ref[:], (((1,), (1,)), ((), ())),
                           preferred_element_type=jnp.float32)
         + b_ref[:])
    h = jnp.maximum(z, 0.0)
    out_ref[:] = (lax.dot_general(h, wlin_ref[:], (((1,), (1,)), ((), ())),
                                  preferred_element_type=jnp.float32)
                  + blin_ref[0])


_BLK = 1024
_GRID = NPAD // _BLK


def _mm1(aggs, deg, x, wl, wr, b):
    return pl.pallas_call(
        _mm1_body,
        grid=(_GRID,),
        in_specs=[
            pl.BlockSpec((2, _BLK, FH), lambda i: (0, i, 0)),
            pl.BlockSpec((2, _BLK, 1), lambda i: (0, i, 0)),
            pl.BlockSpec((_BLK, F), lambda i: (i, 0)),
            pl.BlockSpec((F, F), lambda i: (0, 0)),
            pl.BlockSpec((F, F), lambda i: (0, 0)),
            pl.BlockSpec((1, F), lambda i: (0, 0)),
        ],
        out_specs=pl.BlockSpec((2, _BLK, FH), lambda i: (0, i, 0)),
        out_shape=jax.ShapeDtypeStruct((2, NPAD, FH), jnp.float32),
    )(aggs, deg, x, wl, wr, b)


def _mm2(aggs, deg, hs, wl, wr, b, wlin, blin):
    return pl.pallas_call(
        _mm2_body,
        grid=(_GRID,),
        in_specs=[
            pl.BlockSpec((2, _BLK, FH), lambda i: (0, i, 0)),
            pl.BlockSpec((2, _BLK, 1), lambda i: (0, i, 0)),
            pl.BlockSpec((2, _BLK, FH), lambda i: (0, i, 0)),
            pl.BlockSpec((F, F), lambda i: (0, 0)),
            pl.BlockSpec((F, F), lambda i: (0, 0)),
            pl.BlockSpec((1, F), lambda i: (0, 0)),
            pl.BlockSpec((F, F), lambda i: (0, 0)),
            pl.BlockSpec(memory_space=pltpu.SMEM),
        ],
        out_specs=pl.BlockSpec((_BLK, F), lambda i: (i, 0)),
        out_shape=jax.ShapeDtypeStruct((NPAD, F), jnp.float32),
    )(aggs, deg, hs, wl, wr, b, wlin, blin)


def kernel(x, edge_index, W1l, b1, W1r, W2l, b2, W2r, Wlin, blin):
    src = edge_index[0]
    dst = edge_index[1]
    npad = EPAD - E
    src_p = jnp.concatenate([src, jnp.zeros((npad,), jnp.int32)])
    dst_p = jnp.concatenate(
        [dst, N + (jnp.arange(npad, dtype=jnp.int32) % (NPAD - N))])
    src_r = src_p.reshape(NTILES, NCHUNKS, CHUNK)
    dst_r = dst_p.reshape(NTILES, NCHUNKS, CHUNK)

    xp = jnp.pad(x, ((0, NPAD - N), (0, 0)))
    xs = jnp.stack([xp[:, :FH], xp[:, FH:]])

    agg1, deg = _sc_agg_deg(xs, src_r, dst_r)
    degc = deg[:, :, None]

    h1s = _mm1(agg1, degc, xp, W1l, W1r, b1.reshape(1, F))

    agg2 = _sc_agg_nodeg(h1s, src_r, dst_r)

    wlinp = jnp.pad(Wlin, ((0, F - 1), (0, 0)))
    out = _mm2(agg2, degc, h1s, W2l, W2r, b2.reshape(1, F),
               wlinp, blin)
    return out[:N, :1]


# speedup vs baseline: 1.0593x; 1.0051x over previous
import jax
import jax.numpy as jnp
from jax import lax
from jax.experimental import pallas as pl
from jax.experimental.pallas import tpu as pltpu
from jax.experimental.pallas import tpu_sc as plsc

N = 10000
E = 320000
F = 128
FH = F // 2
NTILES = 16
CHUNK = 128
EPT = 20480
NCHUNKS = EPT // CHUNK
IBLK = 16
NBLOCKS = NCHUNKS // IBLK
GDEPTH = 4
EPAD = EPT * NTILES
NPAD = 10240
ROWS_PER_TILE = NPAD // NTILES


def _make_sc_body(with_deg):
    def body(x_hbm, src_hbm, dst_hbm, *rest):
        if with_deg:
            (agg_hbm, deg_hbm, x_sh, acc_sh, deg_sh, sbuf0, dbuf0,
             sbuf1, dbuf1, gbuf0, gbuf1, gbuf2, gbuf3, zdeg, ones_v,
             gsem, isem, ssem, dsem) = rest
        else:
            (agg_hbm, x_sh, acc_sh, sbuf0, dbuf0,
             sbuf1, dbuf1, gbuf0, gbuf1, gbuf2, gbuf3,
             gsem, isem, ssem) = rest
        gbufs = [gbuf0, gbuf1, gbuf2, gbuf3]
        cid = lax.axis_index("c")
        tid = lax.axis_index("s")

        def zrow(i, _):
            for c in range(4):
                gbuf0[i, pl.ds(c * 16, 16)] = jnp.zeros((16,), jnp.float32)
            return 0
        lax.fori_loop(0, CHUNK, zrow, 0)
        if with_deg:
            for c in range(ROWS_PER_TILE // 16):
                zdeg[pl.ds(c * 16, 16)] = jnp.zeros((16,), jnp.float32)
            for c in range(CHUNK // 16):
                ones_v[pl.ds(c * 16, 16)] = jnp.ones((16,), jnp.float32)

        for b in range(ROWS_PER_TILE // CHUNK):
            pltpu.sync_copy(gbuf0, acc_sh.at[pl.ds(tid * ROWS_PER_TILE
                                                   + b * CHUNK, CHUNK)])
        if with_deg:
            pltpu.sync_copy(zdeg, deg_sh.at[pl.ds(tid * ROWS_PER_TILE,
                                                  ROWS_PER_TILE)])

        pltpu.sync_copy(
            x_hbm.at[cid, pl.ds(tid * ROWS_PER_TILE, ROWS_PER_TILE)],
            x_sh.at[pl.ds(tid * ROWS_PER_TILE, ROWS_PER_TILE)])

        pltpu.sync_copy(src_hbm.at[tid, pl.ds(0, IBLK)], sbuf0)
        pltpu.sync_copy(dst_hbm.at[tid, pl.ds(0, IBLK)], dbuf0)

        plsc.subcore_barrier()

        for k in range(GDEPTH):
            pltpu.async_copy(x_sh.at[sbuf0.at[k]], gbufs[k], gsem)

        for blk in range(NBLOCKS):
            pS, pD = (sbuf0, dbuf0) if blk % 2 == 0 else (sbuf1, dbuf1)
            nS, nD = (sbuf1, dbuf1) if blk % 2 == 0 else (sbuf0, dbuf0)
            if blk + 1 < NBLOCKS:
                ips = pltpu.async_copy(
                    src_hbm.at[tid, pl.ds((blk + 1) * IBLK, IBLK)], nS, isem)
                ipd = pltpu.async_copy(
                    dst_hbm.at[tid, pl.ds((blk + 1) * IBLK, IBLK)], nD, isem)

            deg_core = 0 if blk < NBLOCKS // 2 else 1

            def quad(s, _):
                c = GDEPTH * s
                for k in range(GDEPTH):
                    pltpu.make_async_copy(
                        x_sh.at[pS.at[c + k]], gbufs[k], gsem).wait()
                    pltpu.async_copy(gbufs[k], acc_sh.at[pD.at[c + k]], ssem,
                                     add=True)
                    if with_deg:
                        @pl.when(cid == deg_core)
                        def _(k=k):
                            pltpu.async_copy(ones_v, deg_sh.at[pD.at[c + k]],
                                             dsem, add=True)
                for k in range(GDEPTH):
                    pltpu.make_async_copy(
                        gbufs[k], acc_sh.at[pD.at[c + k]], ssem).wait()

                    @pl.when(s < IBLK // GDEPTH - 1)
                    def _(k=k):
                        pltpu.async_copy(x_sh.at[pS.at[c + GDEPTH + k]],
                                         gbufs[k], gsem)
                return 0
            lax.fori_loop(0, IBLK // GDEPTH, quad, 0)

            if with_deg:
                @pl.when(cid == deg_core)
                def _():
                    def ddrain(i, _):
                        pltpu.make_async_copy(
                            ones_v, deg_sh.at[pD.at[0]], dsem).wait()
                        return 0
                    lax.fori_loop(0, IBLK, ddrain, 0)

            if blk + 1 < NBLOCKS:
                ips.wait()
                ipd.wait()
                for k in range(GDEPTH):
                    pltpu.async_copy(x_sh.at[nS.at[k]], gbufs[k], gsem)

        plsc.subcore_barrier()

        pltpu.sync_copy(
            acc_sh.at[pl.ds(tid * ROWS_PER_TILE, ROWS_PER_TILE)],
            agg_hbm.at[cid, pl.ds(tid * ROWS_PER_TILE, ROWS_PER_TILE)])

        if with_deg:
            pltpu.sync_copy(
                deg_sh.at[pl.ds(tid * ROWS_PER_TILE, ROWS_PER_TILE)],
                deg_hbm.at[cid, pl.ds(tid * ROWS_PER_TILE, ROWS_PER_TILE)])
    return body


def _make_sc_agg(with_deg):
    agg_t = jax.ShapeDtypeStruct((2, NPAD, FH), jnp.float32)
    deg_t = jax.ShapeDtypeStruct((2, NPAD), jnp.float32)
    scratch = [
        pltpu.VMEM_SHARED((NPAD, FH), jnp.float32),
        pltpu.VMEM_SHARED((NPAD, FH), jnp.float32),
    ]
    if with_deg:
        scratch.append(pltpu.VMEM_SHARED((NPAD,), jnp.float32))
    scratch += [
        pltpu.VMEM((IBLK, CHUNK), jnp.int32),
        pltpu.VMEM((IBLK, CHUNK), jnp.int32),
        pltpu.VMEM((IBLK, CHUNK), jnp.int32),
        pltpu.VMEM((IBLK, CHUNK), jnp.int32),
        pltpu.VMEM((CHUNK, FH), jnp.float32),
        pltpu.VMEM((CHUNK, FH), jnp.float32),
        pltpu.VMEM((CHUNK, FH), jnp.float32),
        pltpu.VMEM((CHUNK, FH), jnp.float32),
    ]
    if with_deg:
        scratch += [
            pltpu.VMEM((ROWS_PER_TILE,), jnp.float32),
            pltpu.VMEM((CHUNK,), jnp.float32),
        ]
    scratch += [pltpu.SemaphoreType.DMA, pltpu.SemaphoreType.DMA,
                pltpu.SemaphoreType.DMA]
    if with_deg:
        scratch.append(pltpu.SemaphoreType.DMA)
    return pl.kernel(
        _make_sc_body(with_deg),
        out_type=(agg_t, deg_t) if with_deg else agg_t,
        mesh=plsc.VectorSubcoreMesh(core_axis_name="c", subcore_axis_name="s"),
        scratch_types=scratch,
        compiler_params=pltpu.CompilerParams(use_tc_tiling_on_sc=False),
    )


_sc_agg_deg = _make_sc_agg(True)
_sc_agg_nodeg = _make_sc_agg(False)


def _mmr_body(xs_ref, wr_ref, b_ref, out_ref):
    xfull = jnp.concatenate([xs_ref[0], xs_ref[1]], axis=1)
    out_ref[:] = (lax.dot_general(xfull, wr_ref[:], (((1,), (1,)), ((), ())),
                                  preferred_element_type=jnp.float32)
                  + b_ref[:])


def _mm1_body(aggs_ref, deg_ref, pre_ref, wl_ref, out_ref):
    agg = jnp.concatenate([aggs_ref[0], aggs_ref[1]], axis=1)
    d = jnp.maximum(deg_ref[0] + deg_ref[1], 1.0)
    a = agg / d
    z = (lax.dot_general(a, wl_ref[:], (((1,), (1,)), ((), ())),
                         preferred_element_type=jnp.float32)
         + pre_ref[:])
    h = jnp.maximum(z, 0.0)
    out_ref[0] = h[:, :FH]
    out_ref[1] = h[:, FH:]


def _mm2_body(aggs_ref, deg_ref, pre_ref, wl_ref,
              wlin_ref, blin_ref, out_ref):
    agg = jnp.concatenate([aggs_ref[0], aggs_ref[1]], axis=1)
    d = jnp.maximum(deg_ref[0] + deg_ref[1], 1.0)
    a = agg / d
    z = (lax.dot_general(a, wl_ref[:], (((1,), (1,)), ((), ())),
                         preferred_element_type=jnp.float32)
         + pre_ref[:])
    h = jnp.maximum(z, 0.0)
    out_ref[:] = (lax.dot_general(h, wlin_ref[:], (((1,), (1,)), ((), ())),
                                  preferred_element_type=jnp.float32)
                  + blin_ref[0])


_BLK = 1024
_GRID = NPAD // _BLK


def _mmr(xs, wr, b):
    return pl.pallas_call(
        _mmr_body,
        grid=(_GRID,),
        in_specs=[
            pl.BlockSpec((2, _BLK, FH), lambda i: (0, i, 0)),
            pl.BlockSpec((F, F), lambda i: (0, 0)),
            pl.BlockSpec((1, F), lambda i: (0, 0)),
        ],
        out_specs=pl.BlockSpec((_BLK, F), lambda i: (i, 0)),
        out_shape=jax.ShapeDtypeStruct((NPAD, F), jnp.float32),
    )(xs, wr, b)


def _mm1(aggs, deg, pre, wl):
    return pl.pallas_call(
        _mm1_body,
        grid=(_GRID,),
        in_specs=[
            pl.BlockSpec((2, _BLK, FH), lambda i: (0, i, 0)),
            pl.BlockSpec((2, _BLK, 1), lambda i: (0, i, 0)),
            pl.BlockSpec((_BLK, F), lambda i: (i, 0)),
            pl.BlockSpec((F, F), lambda i: (0, 0)),
        ],
        out_specs=pl.BlockSpec((2, _BLK, FH), lambda i: (0, i, 0)),
        out_shape=jax.ShapeDtypeStruct((2, NPAD, FH), jnp.float32),
    )(aggs, deg, pre, wl)


def _mm2(aggs, deg, pre, wl, wlin, blin):
    return pl.pallas_call(
        _mm2_body,
        grid=(_GRID,),
        in_specs=[
            pl.BlockSpec((2, _BLK, FH), lambda i: (0, i, 0)),
            pl.BlockSpec((2, _BLK, 1), lambda i: (0, i, 0)),
            pl.BlockSpec((_BLK, F), lambda i: (i, 0)),
            pl.BlockSpec((F, F), lambda i: (0, 0)),
            pl.BlockSpec((F, F), lambda i: (0, 0)),
            pl.BlockSpec(memory_space=pltpu.SMEM),
        ],
        out_specs=pl.BlockSpec((_BLK, F), lambda i: (i, 0)),
        out_shape=jax.ShapeDtypeStruct((NPAD, F), jnp.float32),
    )(aggs, deg, pre, wl, wlin, blin)


def kernel(x, edge_index, W1l, b1, W1r, W2l, b2, W2r, Wlin, blin):
    src = edge_index[0]
    dst = edge_index[1]
    npad = EPAD - E
    src_p = jnp.concatenate([src, jnp.zeros((npad,), jnp.int32)])
    dst_p = jnp.concatenate(
        [dst, N + (jnp.arange(npad, dtype=jnp.int32) % (NPAD - N))])
    src_r = src_p.reshape(NTILES, NCHUNKS, CHUNK)
    dst_r = dst_p.reshape(NTILES, NCHUNKS, CHUNK)

    xp = jnp.pad(x, ((0, NPAD - N), (0, 0)))
    xs = jnp.stack([xp[:, :FH], xp[:, FH:]])

    agg1, deg = _sc_agg_deg(xs, src_r, dst_r)
    pre1 = _mmr(xs, W1r, b1.reshape(1, F))
    degc = deg[:, :, None]

    h1s = _mm1(agg1, degc, pre1, W1l)

    agg2 = _sc_agg_nodeg(h1s, src_r, dst_r)
    pre2 = _mmr(h1s, W2r, b2.reshape(1, F))

    wlinp = jnp.pad(Wlin, ((0, F - 1), (0, 0)))
    out = _mm2(agg2, degc, pre2, W2l, wlinp, blin)
    return out[:N, :1]
